# baseline (speedup 1.0000x reference)
import functools

import jax
import jax.numpy as jnp
from jax import lax
from jax.experimental import pallas as pl
from jax.experimental.pallas import tpu as pltpu
from jax.experimental.pallas import tpu_sc as plsc

N = 10000
E = 320000
DIN = 128
DH = 64
NG = 64
NC, NS = 2, 16
NW = NC * NS
CHUNK = 128
CPT = 80
CPT0 = 50
CPT1 = 110
NCH = NS * (CPT0 + CPT1)
E_PAD = NCH * CHUNK
N_PAD = 10240
RPT = N_PAD // NS
DUMMY = N
DW = 16
R = 1024

@functools.cache
def _sc_kernels():
    mesh = plsc.VectorSubcoreMesh(
        core_axis_name="c", subcore_axis_name="s",
        num_cores=NC, num_subcores=NS,
    )

    @functools.partial(
        pl.kernel,
        out_type=jax.ShapeDtypeStruct((NC, N_PAD, DW), jnp.float32),
        mesh=mesh,
        compiler_params=pltpu.CompilerParams(use_tc_tiling_on_sc=False),
        scratch_types=[
            pltpu.VMEM((CPT, CHUNK), jnp.int32),
            pltpu.VMEM((CHUNK, DW), jnp.float32),
            pltpu.VMEM_SHARED((N_PAD, DW), jnp.float32),
        ],
    )
    def _deg(dst_hbm, zeros_hbm, ones_hbm, out_hbm, dst_v, ones_v, deg_sh):
        cid = lax.axis_index("c")
        sid = lax.axis_index("s")
        wid = cid * NS + sid
        row0 = sid * RPT
        pltpu.sync_copy(dst_hbm.at[pl.ds(wid * CPT, CPT)], dst_v)
        pltpu.sync_copy(ones_hbm, ones_v)
        pltpu.sync_copy(
            zeros_hbm.at[pl.ds(row0, RPT)], deg_sh.at[pl.ds(row0, RPT)]
        )
        plsc.subcore_barrier()

        def body(j, carry):
            pltpu.sync_copy(ones_v, deg_sh.at[dst_v.at[j]], add=True)
            return carry

        lax.fori_loop(0, CPT, body, 0)
        plsc.subcore_barrier()
        pltpu.sync_copy(
            deg_sh.at[pl.ds(row0, RPT)], out_hbm.at[cid, pl.ds(row0, RPT)]
        )

    @functools.partial(
        pl.kernel,
        out_type=jax.ShapeDtypeStruct((NC, N_PAD, DH), jnp.float32),
        mesh=mesh,
        compiler_params=pltpu.CompilerParams(use_tc_tiling_on_sc=False),
        scratch_types=[
            pltpu.VMEM((CPT1, CHUNK), jnp.int32),
            pltpu.VMEM((CPT1, CHUNK), jnp.int32),
            pltpu.VMEM((CHUNK, DH), jnp.float32),
            pltpu.VMEM((CHUNK, DH), jnp.float32),
            pltpu.VMEM_SHARED((N_PAD, DH), jnp.float32),
            pltpu.SemaphoreType.DMA,
            pltpu.SemaphoreType.DMA,
            pltpu.SemaphoreType.DMA,
            pltpu.SemaphoreType.DMA,
        ],
    )
    def _agg(y_hbm, src_hbm, dst_hbm, zeros_hbm, out_hbm,
             src_v, dst_v, rows_a, rows_b, agg_sh,
             gsem_a, gsem_b, ssem_a, ssem_b):
        cid = lax.axis_index("c")
        sid = lax.axis_index("s")
        row0 = sid * RPT
        pltpu.sync_copy(
            zeros_hbm.at[pl.ds(row0, RPT)], agg_sh.at[pl.ds(row0, RPT)]
        )
        plsc.subcore_barrier()

        def g_start(j, rows, sem):
            pltpu.async_copy(y_hbm.at[src_v.at[j]], rows, sem)

        def g_wait(j, rows, sem):
            pltpu.make_async_copy(y_hbm.at[src_v.at[j]], rows, sem).wait()

        def s_start(j, rows, sem):
            pltpu.async_copy(rows, agg_sh.at[dst_v.at[j]], sem, add=True)

        def s_wait(j, rows, sem):
            pltpu.make_async_copy(rows, agg_sh.at[dst_v.at[j]], sem).wait()

        def stage_and_run(base, cnt):
            pltpu.sync_copy(
                src_hbm.at[pl.ds(base, cnt)], src_v.at[pl.ds(0, cnt)]
            )
            pltpu.sync_copy(
                dst_hbm.at[pl.ds(base, cnt)], dst_v.at[pl.ds(0, cnt)]
            )
            g_start(0, rows_a, gsem_a)
            g_wait(0, rows_a, gsem_a)
            s_start(0, rows_a, ssem_a)
            g_start(1, rows_b, gsem_b)

            def pair(t, carry):
                j1 = 2 * t + 1
                g_wait(j1, rows_b, gsem_b)
                s_start(j1, rows_b, ssem_b)
                s_wait(j1, rows_a, ssem_a)
                g_start(j1 + 1, rows_a, gsem_a)
                j2 = 2 * t + 2
                g_wait(j2, rows_a, gsem_a)
                s_start(j2, rows_a, ssem_a)
                s_wait(j2, rows_b, ssem_b)
                g_start(j2 + 1, rows_b, gsem_b)
                return carry

            lax.fori_loop(0, (cnt - 2) // 2, pair, 0)
            g_wait(cnt - 1, rows_b, gsem_b)
            s_start(cnt - 1, rows_b, ssem_b)
            s_wait(0, rows_a, ssem_a)
            s_wait(0, rows_b, ssem_b)

        @pl.when(cid == 0)
        def _():
            stage_and_run(sid * CPT0, CPT0)

        @pl.when(cid == 1)
        def _():
            stage_and_run(NS * CPT0 + sid * CPT1, CPT1)

        plsc.subcore_barrier()
        pltpu.sync_copy(
            agg_sh.at[pl.ds(row0, RPT)], out_hbm.at[cid, pl.ds(row0, RPT)]
        )

    return _deg, _agg


def _prep_body(x_ref, w_ref, d0_ref, d1_ref, y_ref, dinv_ref):
    deg = d0_ref[...] + d1_ref[...] + 1.0
    di = lax.rsqrt(deg)
    xw = jnp.dot(x_ref[...], w_ref[...], preferred_element_type=jnp.float32)
    y_ref[...] = xw * di
    dinv_ref[...] = di


_prep = pl.pallas_call(
    _prep_body,
    grid=(N_PAD // R,),
    in_specs=[
        pl.BlockSpec((R, DIN), lambda i: (i, 0)),
        pl.BlockSpec((DIN, DH), lambda i: (0, 0)),
        pl.BlockSpec((R, 1), lambda i: (i, 0)),
        pl.BlockSpec((R, 1), lambda i: (i, 0)),
    ],
    out_specs=[
        pl.BlockSpec((R, DH), lambda i: (i, 0)),
        pl.BlockSpec((R, 1), lambda i: (i, 0)),
    ],
    out_shape=[
        jax.ShapeDtypeStruct((N_PAD, DH), jnp.float32),
        jax.ShapeDtypeStruct((N_PAD, 1), jnp.float32),
    ],
)


def _mid_body(a0_ref, a1_ref, y_ref, dinv_ref, b_ref, w_ref, o_ref):
    di = dinv_ref[...]
    h = (a0_ref[...] + a1_ref[...] + y_ref[...]) * di + b_ref[...]
    h = jnp.maximum(h, 0.0)
    o_ref[...] = jnp.dot(h, w_ref[...], preferred_element_type=jnp.float32) * di


_mid = pl.pallas_call(
    _mid_body,
    grid=(N_PAD // R,),
    in_specs=[
        pl.BlockSpec((R, DH), lambda i: (i, 0)),
        pl.BlockSpec((R, DH), lambda i: (i, 0)),
        pl.BlockSpec((R, DH), lambda i: (i, 0)),
        pl.BlockSpec((R, 1), lambda i: (i, 0)),
        pl.BlockSpec((1, DH), lambda i: (0, 0)),
        pl.BlockSpec((DH, DH), lambda i: (0, 0)),
    ],
    out_specs=pl.BlockSpec((R, DH), lambda i: (i, 0)),
    out_shape=jax.ShapeDtypeStruct((N_PAD, DH), jnp.float32),
)


def _final_body(a0_ref, a1_ref, y_ref, dinv_ref, b_ref, batch_ref, wl_ref,
                bl_ref, o_ref, sums, cnts):
    i = pl.program_id(0)

    @pl.when(i == 0)
    def _():
        sums[...] = jnp.zeros_like(sums)
        cnts[...] = jnp.zeros_like(cnts)

    h = (a0_ref[...] + a1_ref[...] + y_ref[...]) * dinv_ref[...] + b_ref[...]
    gids = lax.broadcasted_iota(jnp.int32, (NG, R), 0)
    mask = (batch_ref[...] == gids).astype(jnp.float32)
    sums[...] += jnp.dot(mask, h, preferred_element_type=jnp.float32)
    cnts[...] += jnp.sum(mask, axis=1, keepdims=True)

    @pl.when(i == pl.num_programs(0) - 1)
    def _():
        g = sums[...] / jnp.maximum(cnts[...], 1.0)
        o_ref[...] = (
            jnp.dot(g, wl_ref[...], preferred_element_type=jnp.float32)
            + bl_ref[...]
        )


_final = pl.pallas_call(
    _final_body,
    grid=(N_PAD // R,),
    in_specs=[
        pl.BlockSpec((R, DH), lambda i: (i, 0)),
        pl.BlockSpec((R, DH), lambda i: (i, 0)),
        pl.BlockSpec((R, DH), lambda i: (i, 0)),
        pl.BlockSpec((R, 1), lambda i: (i, 0)),
        pl.BlockSpec((1, DH), lambda i: (0, 0)),
        pl.BlockSpec((1, R), lambda i: (0, i)),
        pl.BlockSpec((DH, 1), lambda i: (0, 0)),
        pl.BlockSpec((1, 1), lambda i: (0, 0)),
    ],
    out_specs=pl.BlockSpec((NG, 1), lambda i: (0, 0)),
    out_shape=jax.ShapeDtypeStruct((NG, 1), jnp.float32),
    scratch_shapes=[
        pltpu.VMEM((NG, DH), jnp.float32),
        pltpu.VMEM((NG, 1), jnp.float32),
    ],
)


def kernel(x, edge_index, batch, W1, b1, W2, b2, W3, b3, Wl, bl):
    src = edge_index[0].astype(jnp.int32)
    dst = edge_index[1].astype(jnp.int32)
    pad = E_PAD - E
    src_r = jnp.concatenate([src, jnp.zeros((pad,), jnp.int32)]).reshape(
        NCH, CHUNK
    )
    dst_pad = DUMMY + jnp.arange(pad, dtype=jnp.int32) % (N_PAD - N)
    dst_r = jnp.concatenate([dst, dst_pad]).reshape(NCH, CHUNK)
    x_p = jnp.pad(x, ((0, N_PAD - N), (0, 0)))
    batch_p = jnp.pad(
        batch.astype(jnp.int32), (0, N_PAD - N), constant_values=NG
    ).reshape(1, N_PAD)
    zeros_dw = jnp.zeros((N_PAD, DW), jnp.float32)
    zeros64 = jnp.zeros((N_PAD, DH), jnp.float32)
    ones_dw = jnp.ones((CHUNK, DW), jnp.float32)

    _deg, _agg = _sc_kernels()
    degs = _deg(dst_r, zeros_dw, ones_dw)
    y1, dinv = _prep(x_p, W1, degs[0, :, 0:1], degs[1, :, 0:1])
    a1 = _agg(y1, src_r, dst_r, zeros64)
    y2 = _mid(a1[0], a1[1], y1, dinv, b1.reshape(1, DH), W2)
    a2 = _agg(y2, src_r, dst_r, zeros64)
    y3 = _mid(a2[0], a2[1], y2, dinv, b2.reshape(1, DH), W3)
    a3 = _agg(y3, src_r, dst_r, zeros64)
    out = _final(
        a3[0], a3[1], y3, dinv, b3.reshape(1, DH), batch_p, Wl,
        bl.reshape(1, 1)
    )
    return out


# speedup vs baseline: 1.1888x; 1.1888x over previous
import functools

import jax
import jax.numpy as jnp
from jax import lax
from jax.experimental import pallas as pl
from jax.experimental.pallas import tpu as pltpu
from jax.experimental.pallas import tpu_sc as plsc

N = 10000
E = 320000
DIN = 128
DH = 64
NG = 64
NC, NS = 2, 16
NW = NC * NS
CHUNK = 128
CPT = 80
CPT0 = 110
CPT1 = 50
CPTM = max(CPT0, CPT1)
NCH = NS * (CPT0 + CPT1)
E_PAD = NCH * CHUNK
N_PAD = 10240
RPT = N_PAD // NS
DUMMY = N
DW = 16
R = 1024

@functools.cache
def _sc_kernels():
    mesh = plsc.VectorSubcoreMesh(
        core_axis_name="c", subcore_axis_name="s",
        num_cores=NC, num_subcores=NS,
    )

    @functools.partial(
        pl.kernel,
        out_type=jax.ShapeDtypeStruct((NC, N_PAD, DW), jnp.float32),
        mesh=mesh,
        compiler_params=pltpu.CompilerParams(use_tc_tiling_on_sc=False),
        scratch_types=[
            pltpu.VMEM((CPT, CHUNK), jnp.int32),
            pltpu.VMEM((CHUNK, DW), jnp.float32),
            pltpu.VMEM_SHARED((N_PAD, DW), jnp.float32),
        ],
    )
    def _deg(dst_hbm, zeros_hbm, ones_hbm, out_hbm, dst_v, ones_v, deg_sh):
        cid = lax.axis_index("c")
        sid = lax.axis_index("s")
        wid = cid * NS + sid
        row0 = sid * RPT
        pltpu.sync_copy(dst_hbm.at[pl.ds(wid * CPT, CPT)], dst_v)
        pltpu.sync_copy(ones_hbm, ones_v)
        pltpu.sync_copy(
            zeros_hbm.at[pl.ds(row0, RPT)], deg_sh.at[pl.ds(row0, RPT)]
        )
        plsc.subcore_barrier()

        def body(j, carry):
            pltpu.sync_copy(ones_v, deg_sh.at[dst_v.at[j]], add=True)
            return carry

        lax.fori_loop(0, CPT, body, 0)
        plsc.subcore_barrier()
        pltpu.sync_copy(
            deg_sh.at[pl.ds(row0, RPT)], out_hbm.at[cid, pl.ds(row0, RPT)]
        )

    @functools.partial(
        pl.kernel,
        out_type=jax.ShapeDtypeStruct((NC, N_PAD, DH), jnp.float32),
        mesh=mesh,
        compiler_params=pltpu.CompilerParams(use_tc_tiling_on_sc=False),
        scratch_types=[
            pltpu.VMEM((CPTM, CHUNK), jnp.int32),
            pltpu.VMEM((CPTM, CHUNK), jnp.int32),
            pltpu.VMEM((CHUNK, DH), jnp.float32),
            pltpu.VMEM((CHUNK, DH), jnp.float32),
            pltpu.VMEM_SHARED((N_PAD, DH), jnp.float32),
            pltpu.SemaphoreType.DMA,
            pltpu.SemaphoreType.DMA,
            pltpu.SemaphoreType.DMA,
            pltpu.SemaphoreType.DMA,
        ],
    )
    def _agg(y_hbm, src_hbm, dst_hbm, zeros_hbm, out_hbm,
             src_v, dst_v, rows_a, rows_b, agg_sh,
             gsem_a, gsem_b, ssem_a, ssem_b):
        cid = lax.axis_index("c")
        sid = lax.axis_index("s")
        row0 = sid * RPT
        pltpu.sync_copy(
            zeros_hbm.at[pl.ds(row0, RPT)], agg_sh.at[pl.ds(row0, RPT)]
        )
        plsc.subcore_barrier()

        def g_start(j, rows, sem):
            pltpu.async_copy(y_hbm.at[src_v.at[j]], rows, sem)

        def g_wait(j, rows, sem):
            pltpu.make_async_copy(y_hbm.at[src_v.at[j]], rows, sem).wait()

        def s_start(j, rows, sem):
            pltpu.async_copy(rows, agg_sh.at[dst_v.at[j]], sem, add=True)

        def s_wait(j, rows, sem):
            pltpu.make_async_copy(rows, agg_sh.at[dst_v.at[j]], sem).wait()

        def stage_and_run(base, cnt):
            pltpu.sync_copy(
                src_hbm.at[pl.ds(base, cnt)], src_v.at[pl.ds(0, cnt)]
            )
            pltpu.sync_copy(
                dst_hbm.at[pl.ds(base, cnt)], dst_v.at[pl.ds(0, cnt)]
            )
            g_start(0, rows_a, gsem_a)
            g_wait(0, rows_a, gsem_a)
            s_start(0, rows_a, ssem_a)
            g_start(1, rows_b, gsem_b)

            def pair(t, carry):
                j1 = 2 * t + 1
                g_wait(j1, rows_b, gsem_b)
                s_start(j1, rows_b, ssem_b)
                s_wait(j1, rows_a, ssem_a)
                g_start(j1 + 1, rows_a, gsem_a)
                j2 = 2 * t + 2
                g_wait(j2, rows_a, gsem_a)
                s_start(j2, rows_a, ssem_a)
                s_wait(j2, rows_b, ssem_b)
                g_start(j2 + 1, rows_b, gsem_b)
                return carry

            lax.fori_loop(0, (cnt - 2) // 2, pair, 0)
            g_wait(cnt - 1, rows_b, gsem_b)
            s_start(cnt - 1, rows_b, ssem_b)
            s_wait(0, rows_a, ssem_a)
            s_wait(0, rows_b, ssem_b)

        @pl.when(cid == 0)
        def _():
            stage_and_run(sid * CPT0, CPT0)

        @pl.when(cid == 1)
        def _():
            stage_and_run(NS * CPT0 + sid * CPT1, CPT1)

        plsc.subcore_barrier()
        pltpu.sync_copy(
            agg_sh.at[pl.ds(row0, RPT)], out_hbm.at[cid, pl.ds(row0, RPT)]
        )

    return _deg, _agg


def _prep_body(x_ref, w_ref, d0_ref, d1_ref, y_ref, dinv_ref):
    deg = d0_ref[...] + d1_ref[...] + 1.0
    di = lax.rsqrt(deg)
    xw = jnp.dot(x_ref[...], w_ref[...], preferred_element_type=jnp.float32)
    y_ref[...] = xw * di
    dinv_ref[...] = di


_prep = pl.pallas_call(
    _prep_body,
    grid=(N_PAD // R,),
    in_specs=[
        pl.BlockSpec((R, DIN), lambda i: (i, 0)),
        pl.BlockSpec((DIN, DH), lambda i: (0, 0)),
        pl.BlockSpec((R, 1), lambda i: (i, 0)),
        pl.BlockSpec((R, 1), lambda i: (i, 0)),
    ],
    out_specs=[
        pl.BlockSpec((R, DH), lambda i: (i, 0)),
        pl.BlockSpec((R, 1), lambda i: (i, 0)),
    ],
    out_shape=[
        jax.ShapeDtypeStruct((N_PAD, DH), jnp.float32),
        jax.ShapeDtypeStruct((N_PAD, 1), jnp.float32),
    ],
)


def _mid_body(a0_ref, a1_ref, y_ref, dinv_ref, b_ref, w_ref, o_ref):
    di = dinv_ref[...]
    h = (a0_ref[...] + a1_ref[...] + y_ref[...]) * di + b_ref[...]
    h = jnp.maximum(h, 0.0)
    o_ref[...] = jnp.dot(h, w_ref[...], preferred_element_type=jnp.float32) * di


_mid = pl.pallas_call(
    _mid_body,
    grid=(N_PAD // R,),
    in_specs=[
        pl.BlockSpec((R, DH), lambda i: (i, 0)),
        pl.BlockSpec((R, DH), lambda i: (i, 0)),
        pl.BlockSpec((R, DH), lambda i: (i, 0)),
        pl.BlockSpec((R, 1), lambda i: (i, 0)),
        pl.BlockSpec((1, DH), lambda i: (0, 0)),
        pl.BlockSpec((DH, DH), lambda i: (0, 0)),
    ],
    out_specs=pl.BlockSpec((R, DH), lambda i: (i, 0)),
    out_shape=jax.ShapeDtypeStruct((N_PAD, DH), jnp.float32),
)


def _final_body(a0_ref, a1_ref, y_ref, dinv_ref, b_ref, batch_ref, wl_ref,
                bl_ref, o_ref, sums, cnts):
    i = pl.program_id(0)

    @pl.when(i == 0)
    def _():
        sums[...] = jnp.zeros_like(sums)
        cnts[...] = jnp.zeros_like(cnts)

    h = (a0_ref[...] + a1_ref[...] + y_ref[...]) * dinv_ref[...] + b_ref[...]
    gids = lax.broadcasted_iota(jnp.int32, (NG, R), 0)
    mask = (batch_ref[...] == gids).astype(jnp.float32)
    sums[...] += jnp.dot(mask, h, preferred_element_type=jnp.float32)
    cnts[...] += jnp.sum(mask, axis=1, keepdims=True)

    @pl.when(i == pl.num_programs(0) - 1)
    def _():
        g = sums[...] / jnp.maximum(cnts[...], 1.0)
        o_ref[...] = (
            jnp.dot(g, wl_ref[...], preferred_element_type=jnp.float32)
            + bl_ref[...]
        )


_final = pl.pallas_call(
    _final_body,
    grid=(N_PAD // R,),
    in_specs=[
        pl.BlockSpec((R, DH), lambda i: (i, 0)),
        pl.BlockSpec((R, DH), lambda i: (i, 0)),
        pl.BlockSpec((R, DH), lambda i: (i, 0)),
        pl.BlockSpec((R, 1), lambda i: (i, 0)),
        pl.BlockSpec((1, DH), lambda i: (0, 0)),
        pl.BlockSpec((1, R), lambda i: (0, i)),
        pl.BlockSpec((DH, 1), lambda i: (0, 0)),
        pl.BlockSpec((1, 1), lambda i: (0, 0)),
    ],
    out_specs=pl.BlockSpec((NG, 1), lambda i: (0, 0)),
    out_shape=jax.ShapeDtypeStruct((NG, 1), jnp.float32),
    scratch_shapes=[
        pltpu.VMEM((NG, DH), jnp.float32),
        pltpu.VMEM((NG, 1), jnp.float32),
    ],
)


def kernel(x, edge_index, batch, W1, b1, W2, b2, W3, b3, Wl, bl):
    src = edge_index[0].astype(jnp.int32)
    dst = edge_index[1].astype(jnp.int32)
    pad = E_PAD - E
    src_r = jnp.concatenate([src, jnp.zeros((pad,), jnp.int32)]).reshape(
        NCH, CHUNK
    )
    dst_pad = DUMMY + jnp.arange(pad, dtype=jnp.int32) % (N_PAD - N)
    dst_r = jnp.concatenate([dst, dst_pad]).reshape(NCH, CHUNK)
    x_p = jnp.pad(x, ((0, N_PAD - N), (0, 0)))
    batch_p = jnp.pad(
        batch.astype(jnp.int32), (0, N_PAD - N), constant_values=NG
    ).reshape(1, N_PAD)
    zeros_dw = jnp.zeros((N_PAD, DW), jnp.float32)
    zeros64 = jnp.zeros((N_PAD, DH), jnp.float32)
    ones_dw = jnp.ones((CHUNK, DW), jnp.float32)

    _deg, _agg = _sc_kernels()
    degs = _deg(dst_r, zeros_dw, ones_dw)
    y1, dinv = _prep(x_p, W1, degs[0, :, 0:1], degs[1, :, 0:1])
    a1 = _agg(y1, src_r, dst_r, zeros64)
    y2 = _mid(a1[0], a1[1], y1, dinv, b1.reshape(1, DH), W2)
    a2 = _agg(y2, src_r, dst_r, zeros64)
    y3 = _mid(a2[0], a2[1], y2, dinv, b2.reshape(1, DH), W3)
    a3 = _agg(y3, src_r, dst_r, zeros64)
    out = _final(
        a3[0], a3[1], y3, dinv, b3.reshape(1, DH), batch_p, Wl,
        bl.reshape(1, 1)
    )
    return out
